# baseline (speedup 1.0000x reference)
import functools

import numpy as np
import jax
import jax.numpy as jnp
from jax import lax
from jax.experimental import pallas as pl
from jax.experimental.pallas import tpu as pltpu
from jax.experimental.pallas import tpu_sc as plsc

_B = 262144
_NUM_SCALES = 16
_MAX_PARAMS = 2 ** 19
_MIN_RES = np.array([16.0, 16.0, 16.0, 16.0])
_MAX_RES = np.array([256.0, 256.0, 256.0, 128.0])
_PRIMES_I32 = [int(np.uint32(p).astype(np.int64) - (2**32 if p > 2**31 else 0))
               for p in (1, 2654435761, 805459861, 3674653429)]
_HASH_MASK = _MAX_PARAMS - 1

_NW = 32
_PW = _B // _NW
_P = 128
_NB = _PW // _P
_G = _P // 16
_K = (_P * 16) // 128


def _level_meta():
    b = np.exp((np.log(_MAX_RES) - np.log(_MIN_RES)) / (_NUM_SCALES - 1))
    levels = []
    total = 0
    for s in range(_NUM_SCALES):
        res = np.ceil(_MIN_RES * np.power(b, s)).astype(np.int64)
        raw = int(res[0] + 1) * int(res[1] + 1) * int(res[2] + 1) * int(res[3] + 1)
        p = raw if raw % 8 == 0 else ((raw + 7) // 8) * 8
        p = min(_MAX_PARAMS, p)
        strides = [1,
                   int(res[0] + 1),
                   int(res[0] + 1) * int(res[1] + 1),
                   int(res[0] + 1) * int(res[1] + 1) * int(res[2] + 1)]
        levels.append(dict(res=[int(r) for r in res],
                           dense=raw <= p,
                           strides=strides,
                           off=total))
        total += p * 2
    return levels


_LEVELS = _level_meta()
_DENSE_LEVELS = [lv for lv in _LEVELS if lv["dense"]]
_HASH_LEVELS = [lv for lv in _LEVELS if not lv["dense"]]
_N_DENSE = len(_DENSE_LEVELS)
_N_HASH = len(_HASH_LEVELS)

_CRES_NP = np.zeros((_N_HASH, 64), np.float32)
_COFF_NP = np.zeros((_N_HASH, 16), np.int32)
for _i, _lv in enumerate(_HASH_LEVELS):
    for _d in range(4):
        _CRES_NP[_i, _d * 16:(_d + 1) * 16] = float(_lv["res"][_d])
    _COFF_NP[_i, :] = _lv["off"]


def _grid_and_frac(coord, resf, res_max_i):
    pos = coord * resf
    gi = jnp.clip(pos.astype(jnp.int32), 0, res_max_i)
    gf = gi.astype(jnp.float32)
    frac = jnp.clip(pos - gf, 0.0, 1.0)
    return gi, frac


def _corner_weights(fr):
    wx = [1.0 - fr[0], fr[0]]
    wy = [1.0 - fr[1], fr[1]]
    wz = [1.0 - fr[2], fr[2]]
    wt = [1.0 - fr[3], fr[3]]
    wxy = [wx[b0] * wy[b1] for b1 in range(2) for b0 in range(2)]
    wzt = [wz[b2] * wt[b3] for b3 in range(2) for b2 in range(2)]
    return wxy, wzt


def _sc_encode_body(xyzts_t, table, cres, coff, out,
                    xyz_v, ia0, ib0, ra0, rb0, ia1, ib1, ra1, rb1,
                    out_v, cres_v, coff_v, sem0, sem1):
    wid = lax.axis_index("s") * 2 + lax.axis_index("c")
    pltpu.sync_copy(cres, cres_v)
    pltpu.sync_copy(coff, coff_v)
    bufs = [(ia0, ib0, ra0, rb0, sem0), (ia1, ib1, ra1, rb1, sem1)]

    def coords(g):
        return [xyz_v[d, pl.ds(g * 16, 16)] for d in range(4)]

    def store_corners(g, terms_a, terms_b, combine, buf):
        idxa_v, idxb_v, rowsa_v, rowsb_v, sem = buf
        for c in range(16):
            t = [terms_b[d] if (c >> d) & 1 else terms_a[d] for d in range(4)]
            h = combine(t)
            idxa_v[pl.ds(256 * g + 16 * c, 16)] = h
            idxb_v[pl.ds(256 * g + 16 * c, 16)] = h + 1

    def fire(buf):
        idxa_v, idxb_v, rowsa_v, rowsb_v, sem = buf
        pltpu.async_copy(table.at[idxa_v], rowsa_v, sem)
        pltpu.async_copy(table.at[idxb_v], rowsb_v, sem)

    def drain(buf):
        idxa_v, idxb_v, rowsa_v, rowsb_v, sem = buf
        pltpu.make_async_copy(table.at[idxa_v], rowsa_v, sem).wait()
        pltpu.make_async_copy(table.at[idxb_v], rowsb_v, sem).wait()

    def accumulate(g, fr, out_row_0, buf):
        _, _, rowsa_v, rowsb_v, _ = buf
        wxy, wzt = _corner_weights(fr)
        f0 = jnp.zeros((16,), jnp.float32)
        f1 = jnp.zeros((16,), jnp.float32)
        for c in range(16):
            v0 = rowsa_v[pl.ds(256 * g + 16 * c, 16)]
            v1 = rowsb_v[pl.ds(256 * g + 16 * c, 16)]
            w = wxy[c & 3] * wzt[(c >> 2) & 3]
            f0 = f0 + w * v0
            f1 = f1 + w * v1
        out_v[out_row_0, pl.ds(g * 16, 16)] = f0
        out_v[out_row_0 + 1, pl.ds(g * 16, 16)] = f1

    def idx_dense(lv, p):
        def body(g, _):
            cs = coords(g)
            ta, tb = [], []
            for d in range(4):
                gi, _fr = _grid_and_frac(cs[d], float(lv["res"][d]),
                                         lv["res"][d] - 1)
                s2 = 2 * lv["strides"][d]
                a = gi * s2
                if d == 0:
                    a = a + lv["off"]
                ta.append(a)
                tb.append(a + s2)
            store_corners(g, ta, tb,
                          lambda t: ((t[0] + t[1]) + (t[2] + t[3])), bufs[p])
            return 0
        lax.fori_loop(0, _G, body, 0)
        fire(bufs[p])

    def acc_dense(lv, li, p):
        def body(g, _):
            cs = coords(g)
            fr = [_grid_and_frac(cs[d], float(lv["res"][d]),
                                 lv["res"][d] - 1)[1] for d in range(4)]
            accumulate(g, fr, 2 * li, bufs[p])
            return 0
        lax.fori_loop(0, _G, body, 0)

    def hash_consts(hl):
        resf = [cres_v[hl, pl.ds(d * 16, 16)] for d in range(4)]
        resi = [rf.astype(jnp.int32) - 1 for rf in resf]
        offv = coff_v[hl, pl.ds(0, 16)]
        return resf, resi, offv

    def idx_hash(hl, p):
        resf, resi, offv = hash_consts(hl)
        def body(g, _):
            cs = coords(g)
            ta, tb = [], []
            for d in range(4):
                gi, _fr = _grid_and_frac(cs[d], resf[d], resi[d])
                a = gi * _PRIMES_I32[d]
                ta.append(a)
                tb.append(a + _PRIMES_I32[d])
            store_corners(
                g, ta, tb,
                lambda t: (((((t[0] ^ t[1]) ^ (t[2] ^ t[3]))
                             & _HASH_MASK) * 2) + offv), bufs[p])
            return 0
        lax.fori_loop(0, _G, body, 0)
        fire(bufs[p])

    def acc_hash(hl, p):
        resf, resi, offv = hash_consts(hl)
        def body(g, _):
            cs = coords(g)
            fr = [_grid_and_frac(cs[d], resf[d], resi[d])[1] for d in range(4)]
            accumulate(g, fr, 2 * _N_DENSE + 2 * hl, bufs[p])
            return 0
        lax.fori_loop(0, _G, body, 0)

    def run_block(blk, carry):
        base = wid * _PW + blk * _P
        pltpu.sync_copy(xyzts_t.at[:, pl.ds(base, _P)], xyz_v)

        idx_dense(_DENSE_LEVELS[0], 0)
        idx_dense(_DENSE_LEVELS[1], 1)
        drain(bufs[0])
        acc_dense(_DENSE_LEVELS[0], 0, 0)
        idx_dense(_DENSE_LEVELS[2], 0)
        drain(bufs[1])
        acc_dense(_DENSE_LEVELS[1], 1, 1)
        idx_hash(0, 1)
        drain(bufs[0])
        acc_dense(_DENSE_LEVELS[2], 2, 0)

        def hash_pair(i, _):
            idx_hash(1 + 2 * i, 0)
            drain(bufs[1])
            acc_hash(2 * i, 1)
            idx_hash(2 + 2 * i, 1)
            drain(bufs[0])
            acc_hash(1 + 2 * i, 0)
            return 0

        lax.fori_loop(0, (_N_HASH - 1) // 2, hash_pair, 0)
        drain(bufs[1])
        acc_hash(_N_HASH - 1, 1)

        pltpu.sync_copy(out_v, out.at[:, pl.ds(base, _P)])
        return carry

    lax.fori_loop(0, _NB, run_block, 0)


@functools.lru_cache(maxsize=1)
def _make_sc_encode():
    return pl.kernel(
        _sc_encode_body,
        out_type=jax.ShapeDtypeStruct((2 * _NUM_SCALES, _B), jnp.float32),
        mesh=plsc.VectorSubcoreMesh(core_axis_name="c", subcore_axis_name="s",
                                    num_cores=2, num_subcores=16),
        compiler_params=pltpu.CompilerParams(needs_layout_passes=False,
                                             use_tc_tiling_on_sc=False),
        scratch_types=[
            pltpu.VMEM((4, _P), jnp.float32),
            pltpu.VMEM((_K * 128,), jnp.int32),
            pltpu.VMEM((_K * 128,), jnp.int32),
            pltpu.VMEM((_K * 128,), jnp.float32),
            pltpu.VMEM((_K * 128,), jnp.float32),
            pltpu.VMEM((_K * 128,), jnp.int32),
            pltpu.VMEM((_K * 128,), jnp.int32),
            pltpu.VMEM((_K * 128,), jnp.float32),
            pltpu.VMEM((_K * 128,), jnp.float32),
            pltpu.VMEM((2 * _NUM_SCALES, _P), jnp.float32),
            pltpu.VMEM((_N_HASH, 64), jnp.float32),
            pltpu.VMEM((_N_HASH, 16), jnp.int32),
            pltpu.SemaphoreType.DMA,
            pltpu.SemaphoreType.DMA,
        ],
    )


def kernel(xyzts, table):
    xyzts_t = xyzts.T
    out_t = _make_sc_encode()(xyzts_t, table,
                              jnp.asarray(_CRES_NP), jnp.asarray(_COFF_NP))
    return out_t.T


# speedup vs baseline: 1.2201x; 1.2201x over previous
import functools

import numpy as np
import jax
import jax.numpy as jnp
from jax import lax
from jax.experimental import pallas as pl
from jax.experimental.pallas import tpu as pltpu
from jax.experimental.pallas import tpu_sc as plsc

_B = 262144
_NUM_SCALES = 16
_MAX_PARAMS = 2 ** 19
_MIN_RES = np.array([16.0, 16.0, 16.0, 16.0])
_MAX_RES = np.array([256.0, 256.0, 256.0, 128.0])
_PRIMES_I32 = [int(np.uint32(p).astype(np.int64) - (2**32 if p > 2**31 else 0))
               for p in (1, 2654435761, 805459861, 3674653429)]
_HASH_MASK = _MAX_PARAMS - 1

_NW = 32
_PW = _B // _NW
_P = 128
_NB = _PW // _P
_G = _P // 16
_K = (_P * 16) // 128


def _level_meta():
    b = np.exp((np.log(_MAX_RES) - np.log(_MIN_RES)) / (_NUM_SCALES - 1))
    levels = []
    total = 0
    for s in range(_NUM_SCALES):
        res = np.ceil(_MIN_RES * np.power(b, s)).astype(np.int64)
        raw = int(res[0] + 1) * int(res[1] + 1) * int(res[2] + 1) * int(res[3] + 1)
        p = raw if raw % 8 == 0 else ((raw + 7) // 8) * 8
        p = min(_MAX_PARAMS, p)
        strides = [1,
                   int(res[0] + 1),
                   int(res[0] + 1) * int(res[1] + 1),
                   int(res[0] + 1) * int(res[1] + 1) * int(res[2] + 1)]
        levels.append(dict(res=[int(r) for r in res],
                           dense=raw <= p,
                           strides=strides,
                           off=total))
        total += p * 2
    return levels


_LEVELS = _level_meta()
_DENSE_LEVELS = [lv for lv in _LEVELS if lv["dense"]]
_DENSE_PREFIX = min(lv["off"] for lv in _LEVELS if not lv["dense"])
_HASH_LEVELS = [lv for lv in _LEVELS if not lv["dense"]]
_N_DENSE = len(_DENSE_LEVELS)
_N_HASH = len(_HASH_LEVELS)

_CRES_NP = np.zeros((_N_HASH, 64), np.float32)
_COFF_NP = np.zeros((_N_HASH, 16), np.int32)
for _i, _lv in enumerate(_HASH_LEVELS):
    for _d in range(4):
        _CRES_NP[_i, _d * 16:(_d + 1) * 16] = float(_lv["res"][_d])
    _COFF_NP[_i, :] = _lv["off"]


def _grid_and_frac(coord, resf, res_max_i):
    pos = coord * resf
    gi = jnp.clip(pos.astype(jnp.int32), 0, res_max_i)
    gf = gi.astype(jnp.float32)
    frac = jnp.clip(pos - gf, 0.0, 1.0)
    return gi, frac


def _corner_weights(fr):
    wx = [1.0 - fr[0], fr[0]]
    wy = [1.0 - fr[1], fr[1]]
    wz = [1.0 - fr[2], fr[2]]
    wt = [1.0 - fr[3], fr[3]]
    wxy = [wx[b0] * wy[b1] for b1 in range(2) for b0 in range(2)]
    wzt = [wz[b2] * wt[b3] for b3 in range(2) for b2 in range(2)]
    return wxy, wzt


def _sc_encode_body(xyzts_t, table, cres, coff, out,
                    xyz_v, ia0, ib0, ra0, rb0, ia1, ib1, ra1, rb1,
                    out_v, cres_v, coff_v, dense_sp, sem0, sem1):
    wid = lax.axis_index("s") * 2 + lax.axis_index("c")
    pltpu.sync_copy(cres, cres_v)
    pltpu.sync_copy(coff, coff_v)
    @pl.when(lax.axis_index("s") == 0)
    def _stage():
        pltpu.sync_copy(table.at[pl.ds(0, _DENSE_PREFIX)], dense_sp)
    plsc.subcore_barrier()
    bufs = [(ia0, ib0, ra0, rb0, sem0), (ia1, ib1, ra1, rb1, sem1)]

    def coords(g):
        return [xyz_v[d, pl.ds(g * 16, 16)] for d in range(4)]

    def store_corners(g, terms_a, terms_b, combine, buf):
        idxa_v, idxb_v, rowsa_v, rowsb_v, sem = buf
        for c in range(16):
            t = [terms_b[d] if (c >> d) & 1 else terms_a[d] for d in range(4)]
            h = combine(t)
            idxa_v[pl.ds(256 * g + 16 * c, 16)] = h
            idxb_v[pl.ds(256 * g + 16 * c, 16)] = h + 1

    def fire(buf, src=None):
        idxa_v, idxb_v, rowsa_v, rowsb_v, sem = buf
        s = table if src is None else src
        pltpu.async_copy(s.at[idxa_v], rowsa_v, sem)
        pltpu.async_copy(s.at[idxb_v], rowsb_v, sem)

    def drain(buf, src=None):
        idxa_v, idxb_v, rowsa_v, rowsb_v, sem = buf
        s = table if src is None else src
        pltpu.make_async_copy(s.at[idxa_v], rowsa_v, sem).wait()
        pltpu.make_async_copy(s.at[idxb_v], rowsb_v, sem).wait()

    def accumulate(g, fr, out_row_0, buf):
        _, _, rowsa_v, rowsb_v, _ = buf
        wxy, wzt = _corner_weights(fr)
        f0 = jnp.zeros((16,), jnp.float32)
        f1 = jnp.zeros((16,), jnp.float32)
        for c in range(16):
            v0 = rowsa_v[pl.ds(256 * g + 16 * c, 16)]
            v1 = rowsb_v[pl.ds(256 * g + 16 * c, 16)]
            w = wxy[c & 3] * wzt[(c >> 2) & 3]
            f0 = f0 + w * v0
            f1 = f1 + w * v1
        out_v[out_row_0, pl.ds(g * 16, 16)] = f0
        out_v[out_row_0 + 1, pl.ds(g * 16, 16)] = f1

    def idx_dense(lv, p):
        def body(g, _):
            cs = coords(g)
            ta, tb = [], []
            for d in range(4):
                gi, _fr = _grid_and_frac(cs[d], float(lv["res"][d]),
                                         lv["res"][d] - 1)
                s2 = 2 * lv["strides"][d]
                a = gi * s2
                if d == 0:
                    a = a + lv["off"]
                ta.append(a)
                tb.append(a + s2)
            store_corners(g, ta, tb,
                          lambda t: ((t[0] + t[1]) + (t[2] + t[3])), bufs[p])
            return 0
        lax.fori_loop(0, _G, body, 0)
        fire(bufs[p], dense_sp)

    def acc_dense(lv, li, p):
        def body(g, _):
            cs = coords(g)
            fr = [_grid_and_frac(cs[d], float(lv["res"][d]),
                                 lv["res"][d] - 1)[1] for d in range(4)]
            accumulate(g, fr, 2 * li, bufs[p])
            return 0
        lax.fori_loop(0, _G, body, 0)

    def hash_consts(hl):
        resf = [cres_v[hl, pl.ds(d * 16, 16)] for d in range(4)]
        resi = [rf.astype(jnp.int32) - 1 for rf in resf]
        offv = coff_v[hl, pl.ds(0, 16)]
        return resf, resi, offv

    def idx_hash(hl, p):
        resf, resi, offv = hash_consts(hl)
        def body(g, _):
            cs = coords(g)
            ta, tb = [], []
            for d in range(4):
                gi, _fr = _grid_and_frac(cs[d], resf[d], resi[d])
                a = gi * _PRIMES_I32[d]
                ta.append(a)
                tb.append(a + _PRIMES_I32[d])
            store_corners(
                g, ta, tb,
                lambda t: (((((t[0] ^ t[1]) ^ (t[2] ^ t[3]))
                             & _HASH_MASK) * 2) + offv), bufs[p])
            return 0
        lax.fori_loop(0, _G, body, 0)
        fire(bufs[p])

    def acc_hash(hl, p):
        resf, resi, offv = hash_consts(hl)
        def body(g, _):
            cs = coords(g)
            fr = [_grid_and_frac(cs[d], resf[d], resi[d])[1] for d in range(4)]
            accumulate(g, fr, 2 * _N_DENSE + 2 * hl, bufs[p])
            return 0
        lax.fori_loop(0, _G, body, 0)

    def run_block(blk, carry):
        base = wid * _PW + blk * _P
        pltpu.sync_copy(xyzts_t.at[:, pl.ds(base, _P)], xyz_v)

        idx_dense(_DENSE_LEVELS[0], 0)
        idx_dense(_DENSE_LEVELS[1], 1)
        drain(bufs[0], dense_sp)
        acc_dense(_DENSE_LEVELS[0], 0, 0)
        idx_dense(_DENSE_LEVELS[2], 0)
        drain(bufs[1], dense_sp)
        acc_dense(_DENSE_LEVELS[1], 1, 1)
        idx_hash(0, 1)
        drain(bufs[0], dense_sp)
        acc_dense(_DENSE_LEVELS[2], 2, 0)

        def hash_pair(i, _):
            idx_hash(1 + 2 * i, 0)
            drain(bufs[1])
            acc_hash(2 * i, 1)
            idx_hash(2 + 2 * i, 1)
            drain(bufs[0])
            acc_hash(1 + 2 * i, 0)
            return 0

        lax.fori_loop(0, (_N_HASH - 1) // 2, hash_pair, 0)
        drain(bufs[1])
        acc_hash(_N_HASH - 1, 1)

        pltpu.sync_copy(out_v, out.at[:, pl.ds(base, _P)])
        return carry

    lax.fori_loop(0, _NB, run_block, 0)


@functools.lru_cache(maxsize=1)
def _make_sc_encode():
    return pl.kernel(
        _sc_encode_body,
        out_type=jax.ShapeDtypeStruct((2 * _NUM_SCALES, _B), jnp.float32),
        mesh=plsc.VectorSubcoreMesh(core_axis_name="c", subcore_axis_name="s",
                                    num_cores=2, num_subcores=16),
        compiler_params=pltpu.CompilerParams(needs_layout_passes=False,
                                             use_tc_tiling_on_sc=False),
        scratch_types=[
            pltpu.VMEM((4, _P), jnp.float32),
            pltpu.VMEM((_K * 128,), jnp.int32),
            pltpu.VMEM((_K * 128,), jnp.int32),
            pltpu.VMEM((_K * 128,), jnp.float32),
            pltpu.VMEM((_K * 128,), jnp.float32),
            pltpu.VMEM((_K * 128,), jnp.int32),
            pltpu.VMEM((_K * 128,), jnp.int32),
            pltpu.VMEM((_K * 128,), jnp.float32),
            pltpu.VMEM((_K * 128,), jnp.float32),
            pltpu.VMEM((2 * _NUM_SCALES, _P), jnp.float32),
            pltpu.VMEM((_N_HASH, 64), jnp.float32),
            pltpu.VMEM((_N_HASH, 16), jnp.int32),
            pltpu.VMEM_SHARED((_DENSE_PREFIX,), jnp.float32),
            pltpu.SemaphoreType.DMA,
            pltpu.SemaphoreType.DMA,
        ],
    )


def kernel(xyzts, table):
    xyzts_t = xyzts.T
    out_t = _make_sc_encode()(xyzts_t, table,
                              jnp.asarray(_CRES_NP), jnp.asarray(_COFF_NP))
    return out_t.T
